# baseline (speedup 1.0000x reference)
import functools

import jax
import jax.numpy as jnp
from jax import lax
from jax.experimental import pallas as pl
from jax.experimental.pallas import tpu as pltpu
from jax.experimental.pallas import tpu_sc as plsc

N = 100000
DIM_IN = 128
DIM_PE = 32
DIM_H = 96

NUM_WORKERS = 32
CHUNK = 128
CHUNKS_PER_WORKER = 25
B_PER_W = CHUNK * CHUNKS_PER_WORKER
N_PAD = NUM_WORKERS * B_PER_W


def _sc_gather(z1d, table):
    mesh = plsc.VectorSubcoreMesh(core_axis_name="c", subcore_axis_name="s")

    @functools.partial(
        pl.kernel,
        out_type=jax.ShapeDtypeStruct((N_PAD, DIM_PE), jnp.float32),
        mesh=mesh,
        scratch_types=[
            pltpu.VMEM((B_PER_W,), jnp.int32),
            pltpu.VMEM((B_PER_W, DIM_PE), jnp.float32),
            pltpu.SemaphoreType.DMA,
        ],
        compiler_params=pltpu.CompilerParams(use_tc_tiling_on_sc=False),
    )
    def k(z_hbm, table_hbm, out_hbm, idx_v, rows_v, sem):
        wid = lax.axis_index("s") * 2 + lax.axis_index("c")
        pltpu.sync_copy(z_hbm.at[pl.ds(wid * B_PER_W, B_PER_W)], idx_v)

        pltpu.async_copy(table_hbm.at[idx_v], rows_v, sem).wait()
        pltpu.sync_copy(rows_v, out_hbm.at[pl.ds(wid * B_PER_W, B_PER_W)])

    return k(z1d, table)


def _tc_body(x_ref, emb_ref, w_ref, b_ref, out_ref):
    h = jnp.dot(x_ref[...], w_ref[...], preferred_element_type=jnp.float32)
    h = h + b_ref[...]
    out_ref[...] = jnp.concatenate([h, emb_ref[...]], axis=1)


def _tc_fused(x, z_emb, W, b2d, block_rows):
    grid = (N // block_rows,)
    return pl.pallas_call(
        _tc_body,
        grid=grid,
        in_specs=[
            pl.BlockSpec((block_rows, DIM_IN), lambda i: (i, 0)),
            pl.BlockSpec((block_rows, DIM_PE), lambda i: (i, 0)),
            pl.BlockSpec((DIM_IN, DIM_H), lambda i: (0, 0)),
            pl.BlockSpec((1, DIM_H), lambda i: (0, 0)),
        ],
        out_specs=pl.BlockSpec((block_rows, DIM_IN), lambda i: (i, 0)),
        out_shape=jax.ShapeDtypeStruct((N, DIM_IN), jnp.float32),
    )(x, z_emb, W, b2d)


def kernel(x, z, table, W, b):
    z = z.astype(jnp.int32)
    z_pad = jnp.concatenate([z, jnp.zeros((N_PAD - N,), jnp.int32)])
    z_emb = _sc_gather(z_pad, table)
    return _tc_fused(x, z_emb, W, b.reshape(1, DIM_H), block_rows=2000)


# speedup vs baseline: 1.0968x; 1.0968x over previous
import functools

import jax
import jax.numpy as jnp
from jax import lax
from jax.experimental import pallas as pl
from jax.experimental.pallas import tpu as pltpu
from jax.experimental.pallas import tpu_sc as plsc

N = 100000
DIM_IN = 128
DIM_PE = 32
DIM_H = 96

NUM_WORKERS = 32
CHUNK = 128
CHUNKS_PER_WORKER = 25
B_PER_W = CHUNK * CHUNKS_PER_WORKER
N_PAD = NUM_WORKERS * B_PER_W


def _sc_gather(z1d, table):
    mesh = plsc.VectorSubcoreMesh(core_axis_name="c", subcore_axis_name="s")

    @functools.partial(
        pl.kernel,
        out_type=jax.ShapeDtypeStruct((N_PAD, DIM_PE), jnp.float32),
        mesh=mesh,
        scratch_types=[
            pltpu.VMEM((B_PER_W,), jnp.int32),
            pltpu.VMEM((B_PER_W, DIM_PE), jnp.float32),
            pltpu.SemaphoreType.DMA,
        ],
        compiler_params=pltpu.CompilerParams(use_tc_tiling_on_sc=False),
    )
    def k(z_hbm, table_hbm, out_hbm, idx_v, rows_v, sem):
        wid = lax.axis_index("s") * 2 + lax.axis_index("c")
        pltpu.sync_copy(z_hbm.at[pl.ds(wid * B_PER_W, B_PER_W)], idx_v)

        pltpu.async_copy(table_hbm.at[idx_v], rows_v, sem).wait()
        pltpu.sync_copy(rows_v, out_hbm.at[pl.ds(wid * B_PER_W, B_PER_W)])

    return k(z1d, table)


def _tc_body(x_ref, emb_ref, w_ref, b_ref, out_ref):
    h = jnp.dot(x_ref[...], w_ref[...], preferred_element_type=jnp.float32)
    h = h + b_ref[...]
    out_ref[...] = jnp.concatenate([h, emb_ref[...]], axis=1)


def _tc_fused(x, z_emb, W, b2d, block_rows):
    grid = (N // block_rows,)
    return pl.pallas_call(
        _tc_body,
        grid=grid,
        in_specs=[
            pl.BlockSpec((block_rows, DIM_IN), lambda i: (i, 0)),
            pl.BlockSpec((block_rows, DIM_PE), lambda i: (i, 0)),
            pl.BlockSpec((DIM_IN, DIM_H), lambda i: (0, 0)),
            pl.BlockSpec((1, DIM_H), lambda i: (0, 0)),
        ],
        out_specs=pl.BlockSpec((block_rows, DIM_IN), lambda i: (i, 0)),
        out_shape=jax.ShapeDtypeStruct((N, DIM_IN), jnp.float32),
    )(x, z_emb, W, b2d)


def kernel(x, z, table, W, b):
    z = z.astype(jnp.int32)
    z_pad = jnp.concatenate([z, jnp.zeros((N_PAD - N,), jnp.int32)])
    z_emb = _sc_gather(z_pad, table)
    return _tc_fused(x, z_emb, W, b.reshape(1, DIM_H), block_rows=4000)


# speedup vs baseline: 1.1197x; 1.0209x over previous
import functools

import jax
import jax.numpy as jnp
from jax import lax
from jax.experimental import pallas as pl
from jax.experimental.pallas import tpu as pltpu
from jax.experimental.pallas import tpu_sc as plsc

N = 100000
DIM_IN = 128
DIM_PE = 32
DIM_H = 96

NUM_WORKERS = 32
CHUNK = 128
CHUNKS_PER_WORKER = 25
B_PER_W = CHUNK * CHUNKS_PER_WORKER
N_PAD = NUM_WORKERS * B_PER_W


def _sc_gather(z1d, table):
    mesh = plsc.VectorSubcoreMesh(core_axis_name="c", subcore_axis_name="s")

    @functools.partial(
        pl.kernel,
        out_type=jax.ShapeDtypeStruct((N_PAD, DIM_PE), jnp.float32),
        mesh=mesh,
        scratch_types=[
            pltpu.VMEM((B_PER_W,), jnp.int32),
            pltpu.VMEM((B_PER_W, DIM_PE), jnp.float32),
            pltpu.SemaphoreType.DMA,
        ],
        compiler_params=pltpu.CompilerParams(use_tc_tiling_on_sc=False),
    )
    def k(z_hbm, table_hbm, out_hbm, idx_v, rows_v, sem):
        wid = lax.axis_index("s") * 2 + lax.axis_index("c")
        pltpu.sync_copy(z_hbm.at[pl.ds(wid * B_PER_W, B_PER_W)], idx_v)

        pltpu.async_copy(table_hbm.at[idx_v], rows_v, sem).wait()
        pltpu.sync_copy(rows_v, out_hbm.at[pl.ds(wid * B_PER_W, B_PER_W)])

    return k(z1d, table)


def _tc_body(x_ref, emb_ref, w_ref, b_ref, out_ref):
    h = jnp.dot(x_ref[...], w_ref[...], preferred_element_type=jnp.float32)
    h = h + b_ref[...]
    out_ref[...] = jnp.concatenate([h, emb_ref[...]], axis=1)


def _tc_fused(x, z_emb, W, b2d, block_rows):
    grid = (N // block_rows,)
    return pl.pallas_call(
        _tc_body,
        grid=grid,
        in_specs=[
            pl.BlockSpec((block_rows, DIM_IN), lambda i: (i, 0)),
            pl.BlockSpec((block_rows, DIM_PE), lambda i: (i, 0)),
            pl.BlockSpec((DIM_IN, DIM_H), lambda i: (0, 0)),
            pl.BlockSpec((1, DIM_H), lambda i: (0, 0)),
        ],
        out_specs=pl.BlockSpec((block_rows, DIM_IN), lambda i: (i, 0)),
        out_shape=jax.ShapeDtypeStruct((N, DIM_IN), jnp.float32),
    )(x, z_emb, W, b2d)


def kernel(x, z, table, W, b):
    z = z.astype(jnp.int32)
    z_pad = jnp.concatenate([z, jnp.zeros((N_PAD - N,), jnp.int32)])
    z_emb = _sc_gather(z_pad, table)
    return _tc_fused(x, z_emb, W, b.reshape(1, DIM_H), block_rows=10000)
